# baseline (speedup 1.0000x reference)
import functools

import jax
import jax.numpy as jnp
from jax import lax
from jax.experimental import pallas as pl
from jax.experimental.pallas import tpu as pltpu
from jax.experimental.pallas import tpu_sc as plsc

VOCAB = 1000000
DIM = 64
BATCH = 16384
CHUNK = 128
WBLK = 16384


@functools.cache
def _make_gather(V, D, B):
    info = plsc.get_sparse_core_info()
    NC, NS = info.num_cores, info.num_subcores
    NW = NC * NS
    b_per_w = B // NW
    n_chunks = b_per_w // CHUNK
    mesh = plsc.VectorSubcoreMesh(core_axis_name="c", subcore_axis_name="s")

    @functools.partial(
        pl.kernel,
        mesh=mesh,
        compiler_params=pltpu.CompilerParams(use_tc_tiling_on_sc=False),
        out_type=jax.ShapeDtypeStruct((B, D), jnp.float32),
        scratch_types=[
            pltpu.VMEM((n_chunks, CHUNK), jnp.int32),
            pltpu.VMEM((b_per_w, D), jnp.float32),
            pltpu.SemaphoreType.DMA,
        ],
    )
    def gather_kernel(idx_hbm, table_hbm, dummy_hbm, out_e, idx_v, rows_v,
                      gsem):
        del dummy_hbm
        wid = lax.axis_index("s") * NC + lax.axis_index("c")
        base = wid * b_per_w

        pltpu.sync_copy(idx_hbm.at[wid], idx_v)
        gathers = [
            pltpu.async_copy(
                table_hbm.at[idx_v.at[j]],
                rows_v.at[pl.ds(j * CHUNK, CHUNK)],
                gsem,
            )
            for j in range(n_chunks)
        ]
        for g in gathers:
            g.wait()
        pltpu.sync_copy(rows_v, out_e.at[pl.ds(base, b_per_w)])

    return gather_kernel


def _copy_block(src_ref, dst_ref, tick_ref):
    dst_ref[...] = src_ref[...]
    tick_ref[...] = jnp.zeros_like(tick_ref)


@functools.cache
def _make_wcopy(V, D):
    n_blocks = -(-V // WBLK)
    return pl.pallas_call(
        _copy_block,
        grid=(n_blocks,),
        in_specs=[pl.BlockSpec((D, WBLK), lambda i: (0, i))],
        out_specs=[
            pl.BlockSpec((D, WBLK), lambda i: (0, i)),
            pl.BlockSpec((8, 128), lambda i: (0, 0)),
        ],
        out_shape=[
            jax.ShapeDtypeStruct((D, V), jnp.float32),
            jax.ShapeDtypeStruct((8, 128), jnp.float32),
        ],
    )


def kernel(inputs, embedding_table, nce_weights, nce_biases):
    info = plsc.get_sparse_core_info()
    NW = info.num_cores * info.num_subcores
    idx3 = inputs.reshape(NW, BATCH // NW // CHUNK, CHUNK)
    w_t, tick = _make_wcopy(VOCAB, DIM)(nce_weights.T)
    embed = _make_gather(VOCAB, DIM, BATCH)(idx3, embedding_table, tick)
    return (embed, w_t.T, nce_biases)


# speedup vs baseline: 1.0058x; 1.0058x over previous
import functools

import jax
import jax.numpy as jnp
from jax import lax
from jax.experimental import pallas as pl
from jax.experimental.pallas import tpu as pltpu
from jax.experimental.pallas import tpu_sc as plsc

VOCAB = 1000000
DIM = 64
BATCH = 16384
CHUNK = 128
WBLK = 65536


@functools.cache
def _make_gather(V, D, B):
    info = plsc.get_sparse_core_info()
    NC, NS = info.num_cores, info.num_subcores
    NW = NC * NS
    b_per_w = B // NW
    n_chunks = b_per_w // CHUNK
    mesh = plsc.VectorSubcoreMesh(core_axis_name="c", subcore_axis_name="s")

    @functools.partial(
        pl.kernel,
        mesh=mesh,
        compiler_params=pltpu.CompilerParams(use_tc_tiling_on_sc=False),
        out_type=jax.ShapeDtypeStruct((B, D), jnp.float32),
        scratch_types=[
            pltpu.VMEM((n_chunks, CHUNK), jnp.int32),
            pltpu.VMEM((b_per_w, D), jnp.float32),
            pltpu.SemaphoreType.DMA,
        ],
    )
    def gather_kernel(idx_hbm, table_hbm, dummy_hbm, out_e, idx_v, rows_v,
                      gsem):
        del dummy_hbm
        wid = lax.axis_index("s") * NC + lax.axis_index("c")
        base = wid * b_per_w

        pltpu.sync_copy(idx_hbm.at[wid], idx_v)
        gathers = [
            pltpu.async_copy(
                table_hbm.at[idx_v.at[j]],
                rows_v.at[pl.ds(j * CHUNK, CHUNK)],
                gsem,
            )
            for j in range(n_chunks)
        ]
        for g in gathers:
            g.wait()
        pltpu.sync_copy(rows_v, out_e.at[pl.ds(base, b_per_w)])

    return gather_kernel


def _copy_block(src_ref, dst_ref, tick_ref):
    dst_ref[...] = src_ref[...]
    tick_ref[...] = jnp.zeros_like(tick_ref)


@functools.cache
def _make_wcopy(V, D):
    n_blocks = -(-V // WBLK)
    return pl.pallas_call(
        _copy_block,
        grid=(D // 8, n_blocks),
        in_specs=[pl.BlockSpec((8, WBLK), lambda r, c: (r, c))],
        out_specs=[
            pl.BlockSpec((8, WBLK), lambda r, c: (r, c)),
            pl.BlockSpec((8, 128), lambda r, c: (0, 0)),
        ],
        out_shape=[
            jax.ShapeDtypeStruct((D, V), jnp.float32),
            jax.ShapeDtypeStruct((8, 128), jnp.float32),
        ],
    )


def kernel(inputs, embedding_table, nce_weights, nce_biases):
    info = plsc.get_sparse_core_info()
    NW = info.num_cores * info.num_subcores
    idx3 = inputs.reshape(NW, BATCH // NW // CHUNK, CHUNK)
    w_t, tick = _make_wcopy(VOCAB, DIM)(nce_weights.T)
    embed = _make_gather(VOCAB, DIM, BATCH)(idx3, embedding_table, tick)
    return (embed, w_t.T, nce_biases)
